# Initial kernel scaffold; baseline (speedup 1.0000x reference)
import jax
import jax.numpy as jnp
from jax.experimental import pallas as pl


def kernel(x, embedding_table):
    raise NotImplementedError("write your pallas kernel here")


# speedup vs baseline: 1.0148x; 1.0148x over previous
import functools

import jax
import jax.numpy as jnp
from jax import lax
from jax.experimental import pallas as pl
from jax.experimental.pallas import tpu as pltpu
from jax.experimental.pallas import tpu_sc as plsc

_NW = 32


@functools.partial(jax.jit, static_argnums=(2, 3, 4))
def _gather_rows(idx3, table, n_chunks, chunk, d):
    n = _NW * n_chunks * chunk
    mesh = plsc.VectorSubcoreMesh(core_axis_name="c", subcore_axis_name="s")

    @functools.partial(
        pl.kernel,
        mesh=mesh,
        out_type=jax.ShapeDtypeStruct((n, d), jnp.float32),
        scratch_types=[
            pltpu.VMEM((n_chunks, chunk), jnp.int32),
            pltpu.VMEM((chunk, d), jnp.float32),
            pltpu.SemaphoreType.DMA,
        ],
        compiler_params=pltpu.CompilerParams(use_tc_tiling_on_sc=False),
    )
    def gather_kernel(idx_hbm, tab_hbm, out_hbm, idx_v, rows_v, sem):
        cid = lax.axis_index("c")
        sid = lax.axis_index("s")
        wid = sid * 2 + cid
        base = wid * (n_chunks * chunk)
        pltpu.sync_copy(idx_hbm.at[wid], idx_v)

        def body(j, carry):
            pltpu.async_copy(tab_hbm.at[idx_v.at[j]], rows_v, sem).wait()
            pltpu.sync_copy(rows_v, out_hbm.at[pl.ds(base + j * chunk, chunk)])
            return carry

        lax.fori_loop(0, n_chunks, body, 0)

    return gather_kernel(idx3, table)


def kernel(x, embedding_table):
    b, s = x.shape
    v, d = embedding_table.shape
    n = b * s
    chunk = 64
    n_chunks = n // (_NW * chunk)
    idx3 = x.reshape(_NW, n_chunks, chunk).astype(jnp.int32)
    out = _gather_rows(idx3, embedding_table, n_chunks, chunk, d)
    return out.reshape(b, s, d)


# speedup vs baseline: 1.0341x; 1.0190x over previous
import functools

import jax
import jax.numpy as jnp
from jax import lax
from jax.experimental import pallas as pl
from jax.experimental.pallas import tpu as pltpu
from jax.experimental.pallas import tpu_sc as plsc

_NW = 32


@functools.partial(jax.jit, static_argnums=(2, 3, 4))
def _gather_rows(idx3, table, n_chunks, chunk, d):
    n = _NW * n_chunks * chunk
    mesh = plsc.VectorSubcoreMesh(core_axis_name="c", subcore_axis_name="s")

    @functools.partial(
        pl.kernel,
        mesh=mesh,
        out_type=jax.ShapeDtypeStruct((n, d), jnp.float32),
        scratch_types=[
            pltpu.VMEM((n_chunks, chunk), jnp.int32),
            pltpu.VMEM((chunk, d), jnp.float32),
            pltpu.VMEM((chunk, d), jnp.float32),
            pltpu.SemaphoreType.DMA,
            pltpu.SemaphoreType.DMA,
            pltpu.SemaphoreType.DMA,
            pltpu.SemaphoreType.DMA,
        ],
        compiler_params=pltpu.CompilerParams(use_tc_tiling_on_sc=False),
    )
    def gather_kernel(idx_hbm, tab_hbm, out_hbm, idx_v, rows0, rows1, g0, g1, s0, s1):
        cid = lax.axis_index("c")
        sid = lax.axis_index("s")
        wid = sid * 2 + cid
        base = wid * (n_chunks * chunk)
        rows = (rows0, rows1)
        gsem = (g0, g1)
        ssem = (s0, s1)
        pltpu.sync_copy(idx_hbm.at[wid], idx_v)

        def gather_start(j, b):
            pltpu.async_copy(tab_hbm.at[idx_v.at[j]], rows[b], gsem[b])

        def gather_wait(j, b):
            pltpu.make_async_copy(tab_hbm.at[idx_v.at[j]], rows[b], gsem[b]).wait()

        def scatter_start(j, b):
            pltpu.async_copy(rows[b], out_hbm.at[pl.ds(base + j * chunk, chunk)], ssem[b])

        def scatter_wait(b):
            pltpu.make_async_copy(rows[b], out_hbm.at[pl.ds(base, chunk)], ssem[b]).wait()

        gather_start(0, 0)

        def body(jj, carry):
            for b in (0, 1):
                j = jj * 2 + b

                @pl.when(j > 0)
                def _():
                    scatter_wait(1 - b)

                @pl.when(j + 1 < n_chunks)
                def _():
                    gather_start(j + 1, 1 - b)

                gather_wait(j, b)
                scatter_start(j, b)
            return carry

        lax.fori_loop(0, n_chunks // 2, body, 0)
        scatter_wait((n_chunks - 1) % 2)

    return gather_kernel(idx3, table)


def kernel(x, embedding_table):
    b, s = x.shape
    v, d = embedding_table.shape
    n = b * s
    chunk = 40
    n_chunks = n // (_NW * chunk)
    idx3 = x.reshape(_NW, n_chunks, chunk).astype(jnp.int32)
    out = _gather_rows(idx3, embedding_table, n_chunks, chunk, d)
    return out.reshape(b, s, d)


# speedup vs baseline: 1.0368x; 1.0027x over previous
import functools

import jax
import jax.numpy as jnp
from jax import lax
from jax.experimental import pallas as pl
from jax.experimental.pallas import tpu as pltpu
from jax.experimental.pallas import tpu_sc as plsc

_NW = 32


@functools.partial(jax.jit, static_argnums=(2, 3, 4))
def _gather_rows(idx3, table, n_chunks, chunk, d):
    mesh = plsc.VectorSubcoreMesh(core_axis_name="c", subcore_axis_name="s")

    @functools.partial(
        pl.kernel,
        mesh=mesh,
        out_type=jax.ShapeDtypeStruct((_NW * n_chunks, chunk, d), jnp.float32),
        scratch_types=[
            pltpu.VMEM((n_chunks, chunk), jnp.int32),
            pltpu.VMEM((chunk, d), jnp.float32),
            pltpu.VMEM((chunk, d), jnp.float32),
            pltpu.SemaphoreType.DMA,
            pltpu.SemaphoreType.DMA,
            pltpu.SemaphoreType.DMA,
            pltpu.SemaphoreType.DMA,
        ],
        compiler_params=pltpu.CompilerParams(use_tc_tiling_on_sc=False),
    )
    def gather_kernel(idx_hbm, tab_hbm, out_hbm, idx_v, rows0, rows1, g0, g1, s0, s1):
        cid = lax.axis_index("c")
        sid = lax.axis_index("s")
        wid = sid * 2 + cid
        base = wid * n_chunks
        rows = (rows0, rows1)
        gsem = (g0, g1)
        ssem = (s0, s1)
        pltpu.sync_copy(idx_hbm.at[wid], idx_v)

        def gather_start(j, b):
            pltpu.async_copy(tab_hbm.at[idx_v.at[j]], rows[b], gsem[b])

        def gather_wait(j, b):
            pltpu.make_async_copy(tab_hbm.at[idx_v.at[j]], rows[b], gsem[b]).wait()

        def scatter_start(j, b):
            pltpu.async_copy(rows[b], out_hbm.at[base + j], ssem[b])

        def scatter_wait(b):
            pltpu.make_async_copy(rows[b], out_hbm.at[base], ssem[b]).wait()

        gather_start(0, 0)

        def body(jj, carry):
            for b in (0, 1):
                j = jj * 2 + b

                @pl.when(j > 0)
                def _():
                    scatter_wait(1 - b)

                @pl.when(j + 1 < n_chunks)
                def _():
                    gather_start(j + 1, 1 - b)

                gather_wait(j, b)
                scatter_start(j, b)
            return carry

        lax.fori_loop(0, n_chunks // 2, body, 0)
        scatter_wait((n_chunks - 1) % 2)

    return gather_kernel(idx3, table)


def kernel(x, embedding_table):
    b, s = x.shape
    v, d = embedding_table.shape
    chunk = s
    n_chunks = b // _NW
    idx3 = x.reshape(_NW, n_chunks, chunk).astype(jnp.int32)
    return _gather_rows(idx3, embedding_table, n_chunks, chunk, d)


# speedup vs baseline: 1.4131x; 1.3629x over previous
import functools

import jax
import jax.numpy as jnp
from jax import lax
from jax.experimental import pallas as pl
from jax.experimental.pallas import tpu as pltpu
from jax.experimental.pallas import tpu_sc as plsc

_NW = 32
_LANES = 128


@functools.partial(jax.jit, static_argnums=(2, 3))
def _gather_rows(idx1, table_pad, n_chunks, chunk):
    d_pad = table_pad.shape[1]
    n_per_w = n_chunks * chunk
    mesh = plsc.VectorSubcoreMesh(core_axis_name="c", subcore_axis_name="s")

    @functools.partial(
        pl.kernel,
        mesh=mesh,
        out_type=jax.ShapeDtypeStruct((_NW * n_per_w, d_pad), jnp.float32),
        scratch_types=[
            pltpu.VMEM((n_per_w,), jnp.int32),
            pltpu.VMEM((chunk, d_pad), jnp.float32),
            pltpu.VMEM((chunk, d_pad), jnp.float32),
            pltpu.SemaphoreType.DMA,
            pltpu.SemaphoreType.DMA,
            pltpu.SemaphoreType.DMA,
            pltpu.SemaphoreType.DMA,
        ],
    )
    def gather_kernel(idx_hbm, tab_hbm, out_hbm, idx_v, rows0, rows1, g0, g1, s0, s1):
        cid = lax.axis_index("c")
        sid = lax.axis_index("s")
        wid = sid * 2 + cid
        base = wid * n_per_w
        rows = (rows0, rows1)
        gsem = (g0, g1)
        ssem = (s0, s1)
        pltpu.sync_copy(idx_hbm.at[pl.ds(base, n_per_w)], idx_v)

        def gather_start(j, b):
            pltpu.async_copy(
                tab_hbm.at[idx_v.at[pl.ds(j * chunk, chunk)]], rows[b], gsem[b]
            )

        def gather_wait(j, b):
            pltpu.make_async_copy(
                tab_hbm.at[idx_v.at[pl.ds(j * chunk, chunk)]], rows[b], gsem[b]
            ).wait()

        def scatter_start(j, b):
            pltpu.async_copy(rows[b], out_hbm.at[pl.ds(base + j * chunk, chunk)], ssem[b])

        def scatter_wait(b):
            pltpu.make_async_copy(
                rows[b], out_hbm.at[pl.ds(base, chunk)], ssem[b]
            ).wait()

        gather_start(0, 0)

        def body(jj, carry):
            for b in (0, 1):
                j = jj * 2 + b

                @pl.when(j > 0)
                def _():
                    scatter_wait(1 - b)

                gather_wait(j, b)

                @pl.when(j + 1 < n_chunks)
                def _():
                    gather_start(j + 1, 1 - b)

                scatter_start(j, b)
            return carry

        lax.fori_loop(0, n_chunks // 2, body, 0)
        scatter_wait((n_chunks - 1) % 2)

    return gather_kernel(idx1, table_pad)


def kernel(x, embedding_table):
    b, s = x.shape
    v, d = embedding_table.shape
    n = b * s
    chunk = 40
    n_chunks = n // (_NW * chunk)
    d_pad = ((d + _LANES - 1) // _LANES) * _LANES
    table_pad = jnp.pad(embedding_table, ((0, 0), (0, d_pad - d)))
    idx1 = x.reshape(-1).astype(jnp.int32)
    out_pad = _gather_rows(idx1, table_pad, n_chunks, chunk)
    return out_pad[:, :d].reshape(b, s, d)
